# baseline (speedup 1.0000x reference)
import functools

import jax
import jax.numpy as jnp
from jax.experimental import pallas as pl
from jax.experimental.pallas import tpu as pltpu

STATE_DIM = 2048
ACTION_DIM = 512
BATCH = 512

TILE_N = 512

_DNT = (((1,), (1,)), ((), ()))


def _dot_t(x, w):
    return jax.lax.dot_general(
        x.astype(jnp.bfloat16),
        w.astype(jnp.bfloat16),
        _DNT,
        preferred_element_type=jnp.float32,
    )


def _stage1_kernel(state_ref, action_ref, w1s_ref, w1a_ref, b1_ref, h_ref):
    acc = _dot_t(state_ref[...], w1s_ref[...])
    acc += _dot_t(action_ref[...], w1a_ref[...])
    h_ref[...] = jnp.maximum(acc + b1_ref[...][None, :], 0.0)


def _stage2_kernel(h_ref, w2_ref, b2_ref, pn_ref):
    pn_ref[...] = _dot_t(h_ref[...], w2_ref[...]) + b2_ref[...][None, :]


def _stage3_kernel(ns_ref, pn_ref, wf_ref, bf_ref, out_ref):
    i = pl.program_id(0)
    x = jnp.concatenate([ns_ref[...], pn_ref[...]], axis=0)
    f = jnp.maximum(_dot_t(x, wf_ref[...]) + bf_ref[...][None, :], 0.0)
    d = f[:BATCH, :] - f[BATCH:, :]
    partial = jnp.sum(d * d).reshape(1, 1)

    @pl.when(i == 0)
    def _():
        out_ref[...] = jnp.zeros_like(out_ref)

    out_ref[...] += partial


@functools.partial(jax.jit, static_argnames=())
def kernel(state, action, next_state, W1, b1, W2, b2, Wf, bf):
    n_tiles = STATE_DIM // TILE_N

    W1s = W1[:, :STATE_DIM]
    W1a = W1[:, STATE_DIM:]

    h = pl.pallas_call(
        _stage1_kernel,
        grid=(n_tiles,),
        in_specs=[
            pl.BlockSpec((BATCH, STATE_DIM), lambda i: (0, 0)),
            pl.BlockSpec((BATCH, ACTION_DIM), lambda i: (0, 0)),
            pl.BlockSpec((TILE_N, STATE_DIM), lambda i: (i, 0)),
            pl.BlockSpec((TILE_N, ACTION_DIM), lambda i: (i, 0)),
            pl.BlockSpec((TILE_N,), lambda i: (i,)),
        ],
        out_specs=pl.BlockSpec((BATCH, TILE_N), lambda i: (0, i)),
        out_shape=jax.ShapeDtypeStruct((BATCH, STATE_DIM), jnp.float32),
    )(state, action, W1s, W1a, b1)

    pn = pl.pallas_call(
        _stage2_kernel,
        grid=(n_tiles,),
        in_specs=[
            pl.BlockSpec((BATCH, STATE_DIM), lambda i: (0, 0)),
            pl.BlockSpec((TILE_N, STATE_DIM), lambda i: (i, 0)),
            pl.BlockSpec((TILE_N,), lambda i: (i,)),
        ],
        out_specs=pl.BlockSpec((BATCH, TILE_N), lambda i: (0, i)),
        out_shape=jax.ShapeDtypeStruct((BATCH, STATE_DIM), jnp.float32),
    )(h, W2, b2)

    sse = pl.pallas_call(
        _stage3_kernel,
        grid=(n_tiles,),
        in_specs=[
            pl.BlockSpec((BATCH, STATE_DIM), lambda i: (0, 0)),
            pl.BlockSpec((BATCH, STATE_DIM), lambda i: (0, 0)),
            pl.BlockSpec((TILE_N, STATE_DIM), lambda i: (i, 0)),
            pl.BlockSpec((TILE_N,), lambda i: (i,)),
        ],
        out_specs=pl.BlockSpec((1, 1), lambda i: (0, 0)),
        out_shape=jax.ShapeDtypeStruct((1, 1), jnp.float32),
    )(next_state, pn, Wf, bf)

    pred_error = sse[0, 0] / jnp.float32(BATCH * STATE_DIM)
    novelty = jnp.float32(1.0)
    curiosity = pred_error * 0.5 + novelty * 0.5
    return jnp.stack([pred_error, novelty, curiosity])


# speedup vs baseline: 1.1576x; 1.1576x over previous
import functools

import jax
import jax.numpy as jnp
from jax.experimental import pallas as pl
from jax.experimental.pallas import tpu as pltpu

STATE_DIM = 2048
ACTION_DIM = 512
BATCH = 512

TILE_N = 512
N_TILES = STATE_DIM // TILE_N

_DNT = (((1,), (1,)), ((), ()))


def _dot_t(x, w):
    return jax.lax.dot_general(
        x.astype(jnp.bfloat16),
        w.astype(jnp.bfloat16),
        _DNT,
        preferred_element_type=jnp.float32,
    )


def _fused_kernel(
    state_ref, action_ref, ns_ref,
    w1s_ref, w1a_ref, b1_ref,
    w2_ref, b2_ref,
    wf_ref, bf_ref,
    out_ref,
    h_ref, pn_ref,
):
    step = pl.program_id(0)
    j = step % N_TILES
    col = pl.ds(j * TILE_N, TILE_N)

    @pl.when(step < N_TILES)
    def _stage1():
        acc = _dot_t(state_ref[...], w1s_ref[...])
        acc += _dot_t(action_ref[...], w1a_ref[...])
        h_ref[:, col] = jnp.maximum(acc + b1_ref[...][None, :], 0.0)

    @pl.when((step >= N_TILES) & (step < 2 * N_TILES))
    def _stage2():
        pn_ref[:, col] = _dot_t(h_ref[...], w2_ref[...]) + b2_ref[...][None, :]

    @pl.when(step >= 2 * N_TILES)
    def _stage3():
        b = bf_ref[...][None, :]
        fa = jnp.maximum(_dot_t(ns_ref[...], wf_ref[...]) + b, 0.0)
        fp = jnp.maximum(_dot_t(pn_ref[...], wf_ref[...]) + b, 0.0)
        d = fp - fa
        partial = jnp.sum(d * d).reshape(1, 1)

        @pl.when(step == 2 * N_TILES)
        def _():
            out_ref[...] = jnp.zeros_like(out_ref)

        out_ref[...] += partial


def _clamp_tile(lo):
    def index_map(step):
        j = jnp.clip(step - lo, 0, N_TILES - 1)
        return (j, 0)
    return index_map


def _clamp_tile_1d(lo):
    def index_map(step):
        return (jnp.clip(step - lo, 0, N_TILES - 1),)
    return index_map


@functools.partial(jax.jit, static_argnames=())
def kernel(state, action, next_state, W1, b1, W2, b2, Wf, bf):
    W1s = W1[:, :STATE_DIM]
    W1a = W1[:, STATE_DIM:]

    sse = pl.pallas_call(
        _fused_kernel,
        grid=(3 * N_TILES,),
        in_specs=[
            pl.BlockSpec((BATCH, STATE_DIM), lambda step: (0, 0)),
            pl.BlockSpec((BATCH, ACTION_DIM), lambda step: (0, 0)),
            pl.BlockSpec((BATCH, STATE_DIM), lambda step: (0, 0)),
            pl.BlockSpec((TILE_N, STATE_DIM), _clamp_tile(0)),
            pl.BlockSpec((TILE_N, ACTION_DIM), _clamp_tile(0)),
            pl.BlockSpec((TILE_N,), _clamp_tile_1d(0)),
            pl.BlockSpec((TILE_N, STATE_DIM), _clamp_tile(N_TILES)),
            pl.BlockSpec((TILE_N,), _clamp_tile_1d(N_TILES)),
            pl.BlockSpec((TILE_N, STATE_DIM), _clamp_tile(2 * N_TILES)),
            pl.BlockSpec((TILE_N,), _clamp_tile_1d(2 * N_TILES)),
        ],
        out_specs=pl.BlockSpec((1, 1), lambda step: (0, 0)),
        out_shape=jax.ShapeDtypeStruct((1, 1), jnp.float32),
        scratch_shapes=[
            pltpu.VMEM((BATCH, STATE_DIM), jnp.float32),
            pltpu.VMEM((BATCH, STATE_DIM), jnp.float32),
        ],
    )(state, action, next_state, W1s, W1a, b1, W2, b2, Wf, bf)

    pred_error = sse[0, 0] / jnp.float32(BATCH * STATE_DIM)
    novelty = jnp.float32(1.0)
    curiosity = pred_error * 0.5 + novelty * 0.5
    return jnp.stack([pred_error, novelty, curiosity])


# speedup vs baseline: 1.6654x; 1.4386x over previous
import functools

import jax
import jax.numpy as jnp
from jax.experimental import pallas as pl
from jax.experimental.pallas import tpu as pltpu

STATE_DIM = 2048
ACTION_DIM = 512
BATCH = 512

TILE_N = 512
N_TILES = STATE_DIM // TILE_N

_DNT = (((1,), (1,)), ((), ()))


def _dot_t(x_bf16, w_f32):
    return jax.lax.dot_general(
        x_bf16,
        w_f32.astype(jnp.bfloat16),
        _DNT,
        preferred_element_type=jnp.float32,
    )


def _fused_kernel(
    state_ref, action_ref, ns_ref,
    w1s_ref, w1a_ref, b1_ref,
    w2_ref, b2_ref,
    wf_ref, bf_ref,
    out_ref,
    xs_ref, xa_ref, xn_ref, h_ref, pn_ref,
):
    step = pl.program_id(0)
    j = step % N_TILES
    col = pl.ds(j * TILE_N, TILE_N)

    @pl.when(step == 0)
    def _precast():
        xs_ref[...] = state_ref[...].astype(jnp.bfloat16)
        xa_ref[...] = action_ref[...].astype(jnp.bfloat16)
        xn_ref[...] = ns_ref[...].astype(jnp.bfloat16)

    @pl.when(step < N_TILES)
    def _stage1():
        acc = _dot_t(xs_ref[...], w1s_ref[...])
        acc += _dot_t(xa_ref[...], w1a_ref[...])
        h_ref[:, col] = jnp.maximum(acc + b1_ref[...][None, :], 0.0).astype(
            jnp.bfloat16
        )

    @pl.when((step >= N_TILES) & (step < 2 * N_TILES))
    def _stage2():
        pn = _dot_t(h_ref[...], w2_ref[...]) + b2_ref[...][None, :]
        pn_ref[:, col] = pn.astype(jnp.bfloat16)

    @pl.when(step >= 2 * N_TILES)
    def _stage3():
        b = bf_ref[...][None, :]
        wf = wf_ref[...].astype(jnp.bfloat16)
        fa = jnp.maximum(
            jax.lax.dot_general(xn_ref[...], wf, _DNT,
                                preferred_element_type=jnp.float32) + b, 0.0)
        fp = jnp.maximum(
            jax.lax.dot_general(pn_ref[...], wf, _DNT,
                                preferred_element_type=jnp.float32) + b, 0.0)
        d = fp - fa
        partial = jnp.sum(d * d).reshape(1, 1)

        @pl.when(step == 2 * N_TILES)
        def _():
            out_ref[...] = jnp.zeros_like(out_ref)

        out_ref[...] += partial


def _clamp_tile(lo, kblk=0):
    def index_map(step):
        return (jnp.clip(step - lo, 0, N_TILES - 1), kblk)
    return index_map


def _clamp_tile_1d(lo):
    def index_map(step):
        return (jnp.clip(step - lo, 0, N_TILES - 1),)
    return index_map


@functools.partial(jax.jit, static_argnames=())
def kernel(state, action, next_state, W1, b1, W2, b2, Wf, bf):
    sse = pl.pallas_call(
        _fused_kernel,
        grid=(3 * N_TILES,),
        in_specs=[
            pl.BlockSpec((BATCH, STATE_DIM), lambda step: (0, 0)),
            pl.BlockSpec((BATCH, ACTION_DIM), lambda step: (0, 0)),
            pl.BlockSpec((BATCH, STATE_DIM), lambda step: (0, 0)),
            pl.BlockSpec((TILE_N, STATE_DIM), _clamp_tile(0, 0)),
            pl.BlockSpec((TILE_N, ACTION_DIM), _clamp_tile(0, STATE_DIM // ACTION_DIM)),
            pl.BlockSpec((TILE_N,), _clamp_tile_1d(0)),
            pl.BlockSpec((TILE_N, STATE_DIM), _clamp_tile(N_TILES)),
            pl.BlockSpec((TILE_N,), _clamp_tile_1d(N_TILES)),
            pl.BlockSpec((TILE_N, STATE_DIM), _clamp_tile(2 * N_TILES)),
            pl.BlockSpec((TILE_N,), _clamp_tile_1d(2 * N_TILES)),
        ],
        out_specs=pl.BlockSpec((1, 1), lambda step: (0, 0)),
        out_shape=jax.ShapeDtypeStruct((1, 1), jnp.float32),
        scratch_shapes=[
            pltpu.VMEM((BATCH, STATE_DIM), jnp.bfloat16),
            pltpu.VMEM((BATCH, ACTION_DIM), jnp.bfloat16),
            pltpu.VMEM((BATCH, STATE_DIM), jnp.bfloat16),
            pltpu.VMEM((BATCH, STATE_DIM), jnp.bfloat16),
            pltpu.VMEM((BATCH, STATE_DIM), jnp.bfloat16),
        ],
    )(state, action, next_state, W1, W1, b1, W2, b2, Wf, bf)

    pred_error = sse[0, 0] / jnp.float32(BATCH * STATE_DIM)
    novelty = jnp.float32(1.0)
    curiosity = pred_error * 0.5 + novelty * 0.5
    return jnp.stack([pred_error, novelty, curiosity])


# speedup vs baseline: 1.6874x; 1.0132x over previous
import functools

import jax
import jax.numpy as jnp
from jax.experimental import pallas as pl
from jax.experimental.pallas import tpu as pltpu

STATE_DIM = 2048
ACTION_DIM = 512
BATCH = 512

TILE_N = 512
N_TILES = STATE_DIM // TILE_N

_DNT = (((1,), (1,)), ((), ()))


def _dot_t(x, w):
    return jax.lax.dot_general(
        x, w, _DNT,
        precision=jax.lax.Precision.DEFAULT,
        preferred_element_type=jnp.float32,
    )


def _fused_kernel(
    state_ref, action_ref, ns_ref,
    w1s_ref, w1a_ref, b1_ref,
    w2_ref, b2_ref,
    wf_ref, bf_ref,
    out_ref,
    h_ref, pn_ref,
):
    step = pl.program_id(0)
    j = step % N_TILES
    col = pl.ds(j * TILE_N, TILE_N)

    @pl.when(step < N_TILES)
    def _stage1():
        acc = _dot_t(state_ref[...], w1s_ref[...])
        acc += _dot_t(action_ref[...], w1a_ref[...])
        h_ref[:, col] = jnp.maximum(acc + b1_ref[...][None, :], 0.0)

    @pl.when((step >= N_TILES) & (step < 2 * N_TILES))
    def _stage2():
        pn_ref[:, col] = _dot_t(h_ref[...], w2_ref[...]) + b2_ref[...][None, :]

    @pl.when(step >= 2 * N_TILES)
    def _stage3():
        b = bf_ref[...][None, :]
        wf = wf_ref[...]
        fa = jnp.maximum(_dot_t(ns_ref[...], wf) + b, 0.0)
        fp = jnp.maximum(_dot_t(pn_ref[...], wf) + b, 0.0)
        d = fp - fa
        partial = jnp.sum(d * d).reshape(1, 1)

        @pl.when(step == 2 * N_TILES)
        def _():
            out_ref[...] = jnp.zeros_like(out_ref)

        out_ref[...] += partial


def _clamp_tile(lo, kblk=0):
    def index_map(step):
        return (jnp.clip(step - lo, 0, N_TILES - 1), kblk)
    return index_map


def _clamp_tile_1d(lo):
    def index_map(step):
        return (jnp.clip(step - lo, 0, N_TILES - 1),)
    return index_map


@functools.partial(jax.jit, static_argnames=())
def kernel(state, action, next_state, W1, b1, W2, b2, Wf, bf):
    sse = pl.pallas_call(
        _fused_kernel,
        grid=(3 * N_TILES,),
        in_specs=[
            pl.BlockSpec((BATCH, STATE_DIM), lambda step: (0, 0)),
            pl.BlockSpec((BATCH, ACTION_DIM), lambda step: (0, 0)),
            pl.BlockSpec((BATCH, STATE_DIM), lambda step: (0, 0)),
            pl.BlockSpec((TILE_N, STATE_DIM), _clamp_tile(0, 0)),
            pl.BlockSpec((TILE_N, ACTION_DIM), _clamp_tile(0, STATE_DIM // ACTION_DIM)),
            pl.BlockSpec((TILE_N,), _clamp_tile_1d(0)),
            pl.BlockSpec((TILE_N, STATE_DIM), _clamp_tile(N_TILES)),
            pl.BlockSpec((TILE_N,), _clamp_tile_1d(N_TILES)),
            pl.BlockSpec((TILE_N, STATE_DIM), _clamp_tile(2 * N_TILES)),
            pl.BlockSpec((TILE_N,), _clamp_tile_1d(2 * N_TILES)),
        ],
        out_specs=pl.BlockSpec((1, 1), lambda step: (0, 0)),
        out_shape=jax.ShapeDtypeStruct((1, 1), jnp.float32),
        scratch_shapes=[
            pltpu.VMEM((BATCH, STATE_DIM), jnp.float32),
            pltpu.VMEM((BATCH, STATE_DIM), jnp.float32),
        ],
    )(state, action, next_state, W1, W1, b1, W2, b2, Wf, bf)

    pred_error = sse[0, 0] / jnp.float32(BATCH * STATE_DIM)
    novelty = jnp.float32(1.0)
    curiosity = pred_error * 0.5 + novelty * 0.5
    return jnp.stack([pred_error, novelty, curiosity])


# speedup vs baseline: 1.7253x; 1.0225x over previous
import functools

import jax
import jax.numpy as jnp
from jax.experimental import pallas as pl
from jax.experimental.pallas import tpu as pltpu

STATE_DIM = 2048
ACTION_DIM = 512
BATCH = 512

TILE_N = 512
N_TILES = STATE_DIM // TILE_N
KSPLIT = 4
KP = STATE_DIM // KSPLIT

_DNT = (((1,), (1,)), ((), ()))


def _dot_t(x, w):
    return jax.lax.dot_general(
        x, w, _DNT,
        precision=jax.lax.Precision.DEFAULT,
        preferred_element_type=jnp.float32,
    )


def _ksum_dot(x_ref, w_refs):
    acc = _dot_t(x_ref[:, pl.ds(0, KP)], w_refs[0][...])
    for p in range(1, KSPLIT):
        acc += _dot_t(x_ref[:, pl.ds(p * KP, KP)], w_refs[p][...])
    return acc


def _fused_kernel(*refs):
    (state_ref, action_ref, ns_ref) = refs[0:3]
    w1s_refs = refs[3:3 + KSPLIT]
    w1a_ref = refs[3 + KSPLIT]
    b1_ref = refs[4 + KSPLIT]
    w2_refs = refs[5 + KSPLIT:5 + 2 * KSPLIT]
    b2_ref = refs[5 + 2 * KSPLIT]
    wf_refs = refs[6 + 2 * KSPLIT:6 + 3 * KSPLIT]
    bf_ref = refs[6 + 3 * KSPLIT]
    out_ref = refs[7 + 3 * KSPLIT]
    h_ref, pn_ref = refs[8 + 3 * KSPLIT:]

    step = pl.program_id(0)
    j = step % N_TILES
    col = pl.ds(j * TILE_N, TILE_N)

    @pl.when(step < N_TILES)
    def _stage1():
        acc = _ksum_dot(state_ref, w1s_refs)
        acc += _dot_t(action_ref[...], w1a_ref[...])
        h_ref[:, col] = jnp.maximum(acc + b1_ref[...][None, :], 0.0)

    @pl.when((step >= N_TILES) & (step < 2 * N_TILES))
    def _stage2():
        pn_ref[:, col] = _ksum_dot(h_ref, w2_refs) + b2_ref[...][None, :]

    @pl.when(step >= 2 * N_TILES)
    def _stage3():
        b = bf_ref[...][None, :]
        fa = jnp.maximum(_ksum_dot(ns_ref, wf_refs) + b, 0.0)
        fp = jnp.maximum(_ksum_dot(pn_ref, wf_refs) + b, 0.0)
        d = fp - fa
        partial = jnp.sum(d * d).reshape(1, 1)

        @pl.when(step == 2 * N_TILES)
        def _():
            out_ref[...] = jnp.zeros_like(out_ref)

        out_ref[...] += partial


def _clamp_tile(lo, kblk):
    def index_map(step):
        return (jnp.clip(step - lo, 0, N_TILES - 1), kblk)
    return index_map


def _clamp_tile_1d(lo):
    def index_map(step):
        return (jnp.clip(step - lo, 0, N_TILES - 1),)
    return index_map


@functools.partial(jax.jit, static_argnames=())
def kernel(state, action, next_state, W1, b1, W2, b2, Wf, bf):
    in_specs = [
        pl.BlockSpec((BATCH, STATE_DIM), lambda step: (0, 0)),
        pl.BlockSpec((BATCH, ACTION_DIM), lambda step: (0, 0)),
        pl.BlockSpec((BATCH, STATE_DIM), lambda step: (0, 0)),
    ]
    in_specs += [pl.BlockSpec((TILE_N, KP), _clamp_tile(0, p))
                 for p in range(KSPLIT)]
    in_specs += [
        pl.BlockSpec((TILE_N, ACTION_DIM),
                     _clamp_tile(0, STATE_DIM // ACTION_DIM)),
        pl.BlockSpec((TILE_N,), _clamp_tile_1d(0)),
    ]
    in_specs += [pl.BlockSpec((TILE_N, KP), _clamp_tile(N_TILES, p))
                 for p in range(KSPLIT)]
    in_specs += [pl.BlockSpec((TILE_N,), _clamp_tile_1d(N_TILES))]
    in_specs += [pl.BlockSpec((TILE_N, KP), _clamp_tile(2 * N_TILES, p))
                 for p in range(KSPLIT)]
    in_specs += [pl.BlockSpec((TILE_N,), _clamp_tile_1d(2 * N_TILES))]

    operands = (
        [state, action, next_state]
        + [W1] * KSPLIT + [W1, b1]
        + [W2] * KSPLIT + [b2]
        + [Wf] * KSPLIT + [bf]
    )

    sse = pl.pallas_call(
        _fused_kernel,
        grid=(3 * N_TILES,),
        in_specs=in_specs,
        out_specs=pl.BlockSpec((1, 1), lambda step: (0, 0)),
        out_shape=jax.ShapeDtypeStruct((1, 1), jnp.float32),
        scratch_shapes=[
            pltpu.VMEM((BATCH, STATE_DIM), jnp.float32),
            pltpu.VMEM((BATCH, STATE_DIM), jnp.float32),
        ],
    )(*operands)

    pred_error = sse[0, 0] / jnp.float32(BATCH * STATE_DIM)
    novelty = jnp.float32(1.0)
    curiosity = pred_error * 0.5 + novelty * 0.5
    return jnp.stack([pred_error, novelty, curiosity])


# speedup vs baseline: 1.8893x; 1.0951x over previous
import functools

import jax
import jax.numpy as jnp
from jax.experimental import pallas as pl
from jax.experimental.pallas import tpu as pltpu

STATE_DIM = 2048
ACTION_DIM = 512
BATCH = 512

TILE = 512
N_TILES = STATE_DIM // TILE
NSLOTS = 6

_DNT = (((1,), (1,)), ((), ()))

_TILES = [("w1", t) for t in range(N_TILES)] + \
         [("w2", t) for t in range(N_TILES)] + \
         [("wf", t) for t in range(N_TILES)]


def _dot_t(x, w):
    return jax.lax.dot_general(
        x, w, _DNT,
        precision=jax.lax.Precision.DEFAULT,
        preferred_element_type=jnp.float32,
    )


def _body(
    state_ref, action_ref, ns_ref,
    w1_hbm, b1_ref, w2_hbm, b2_ref, wf_hbm, bf_ref,
    out_ref,
    w1a_ref, h_ref, pn_ref, *slot_and_sems,
):
    slots = slot_and_sems[:NSLOTS]
    sems = slot_and_sems[NSLOTS:2 * NSLOTS]
    sem_a = slot_and_sems[2 * NSLOTS]

    def tile_copy(idx, slot):
        kind, t = _TILES[idx]
        rows = pl.ds(t * TILE, TILE)
        if kind == "w1":
            src = w1_hbm.at[rows, pl.ds(0, STATE_DIM)]
        elif kind == "w2":
            src = w2_hbm.at[rows, :]
        else:
            src = wf_hbm.at[rows, :]
        return pltpu.make_async_copy(src, slots[slot], sems[slot])

    cp_a = pltpu.make_async_copy(
        w1_hbm.at[:, pl.ds(STATE_DIM, ACTION_DIM)], w1a_ref, sem_a)
    cp_a.start()
    for i in range(NSLOTS):
        tile_copy(i, i).start()

    sse = jnp.zeros((), jnp.float32)
    for idx in range(len(_TILES)):
        slot = idx % NSLOTS
        tile_copy(idx, slot).wait()
        kind, t = _TILES[idx]
        col = pl.ds(t * TILE, TILE)
        w = slots[slot][...]
        if kind == "w1":
            if t == 0:
                cp_a.wait()
            acc = _dot_t(state_ref[...], w)
            acc += _dot_t(action_ref[...], w1a_ref[pl.ds(t * TILE, TILE), :])
            h_ref[:, col] = jnp.maximum(acc + b1_ref[col][None, :], 0.0)
        elif kind == "w2":
            pn_ref[:, col] = _dot_t(h_ref[...], w) + b2_ref[col][None, :]
        else:
            b = bf_ref[col][None, :]
            fa = jnp.maximum(_dot_t(ns_ref[...], w) + b, 0.0)
            fp = jnp.maximum(_dot_t(pn_ref[...], w) + b, 0.0)
            d = fp - fa
            sse += jnp.sum(d * d)
        nxt = idx + NSLOTS
        if nxt < len(_TILES):
            tile_copy(nxt, slot).start()

    out_ref[...] = sse.reshape(1, 1)


@functools.partial(jax.jit, static_argnames=())
def kernel(state, action, next_state, W1, b1, W2, b2, Wf, bf):
    vmem = functools.partial(pl.BlockSpec, memory_space=pltpu.MemorySpace.VMEM)
    hbm = pl.BlockSpec(memory_space=pltpu.MemorySpace.HBM)
    sse = pl.pallas_call(
        _body,
        in_specs=[
            vmem(), vmem(), vmem(),
            hbm, vmem(),
            hbm, vmem(),
            hbm, vmem(),
        ],
        out_specs=vmem(),
        out_shape=jax.ShapeDtypeStruct((1, 1), jnp.float32),
        scratch_shapes=(
            [pltpu.VMEM((STATE_DIM, ACTION_DIM), jnp.float32)]
            + [pltpu.VMEM((BATCH, STATE_DIM), jnp.float32)] * 2
            + [pltpu.VMEM((TILE, STATE_DIM), jnp.float32)
               for _ in range(NSLOTS)]
            + [pltpu.SemaphoreType.DMA for _ in range(NSLOTS)]
            + [pltpu.SemaphoreType.DMA]
        ),
    )(state, action, next_state, W1, b1, W2, b2, Wf, bf)

    pred_error = sse[0, 0] / jnp.float32(BATCH * STATE_DIM)
    novelty = jnp.float32(1.0)
    curiosity = pred_error * 0.5 + novelty * 0.5
    return jnp.stack([pred_error, novelty, curiosity])
